# baseline (device time: 22200 ns/iter reference)
import jax
import jax.numpy as jnp
from jax import lax
from jax.experimental import pallas as pl
from jax.experimental.pallas import tpu as pltpu

N_DEV = 4

_CLIP = 6.0
_QSCALE = 127.0 / _CLIP
_DEQ = _CLIP / 127.0

_ORDER = (2, 1, 3, 0)


def kernel(x, w_mat):
    m_per, k = x.shape
    n = w_mat.shape[1]
    n_per = n // N_DEV

    def body(x_ref, w_ref, out_ref, send_buf, recv_buf,
             send_sems, recv_sems):
        my = lax.axis_index("i")

        barrier_sem = pltpu.get_barrier_semaphore()
        for d in range(1, N_DEV):
            pl.semaphore_signal(
                barrier_sem, inc=1,
                device_id=((my + d) % N_DEV,),
                device_id_type=pl.DeviceIdType.MESH,
            )

        pl.semaphore_wait(barrier_sem, N_DEV - 1)

        rdmas = []
        for s in range(N_DEV - 1):
            j = (my + _ORDER[s]) % N_DEV
            chunk = jnp.dot(
                x_ref[:, :], w_ref[:, pl.ds(j * n_per, n_per)],
                preferred_element_type=jnp.float32,
            )
            send_buf[s, :, :] = jnp.round(
                jnp.clip(chunk, -_CLIP, _CLIP) * _QSCALE
            ).astype(jnp.int8)
            rdma = pltpu.make_async_remote_copy(
                src_ref=send_buf.at[s],
                dst_ref=recv_buf.at[s],
                send_sem=send_sems.at[s],
                recv_sem=recv_sems.at[s],
                device_id=((my + _ORDER[s]) % N_DEV,),
                device_id_type=pl.DeviceIdType.MESH,
            )
            rdma.start()
            rdmas.append(rdma)

        out_ref[pl.ds(my * m_per, m_per), :] = jnp.dot(
            x_ref[:, :], w_ref[:, pl.ds(my * n_per, n_per)],
            preferred_element_type=jnp.float32,
        )

        for s in (1, 2, 0):
            rdmas[s].wait_recv()
            o = (my - _ORDER[s]) % N_DEV
            out_ref[pl.ds(o * m_per, m_per), :] = (
                recv_buf[s, :, :].astype(jnp.float32) * _DEQ
            )

        for s in range(N_DEV - 1):
            rdmas[s].wait_send()

    return pl.pallas_call(
        body,
        out_shape=jax.ShapeDtypeStruct((N_DEV * m_per, n_per), jnp.float32),
        in_specs=[
            pl.BlockSpec(memory_space=pltpu.VMEM),
            pl.BlockSpec(memory_space=pltpu.VMEM),
        ],
        out_specs=pl.BlockSpec(memory_space=pltpu.VMEM),
        scratch_shapes=[
            pltpu.VMEM((N_DEV - 1, m_per, n_per), jnp.int8),
            pltpu.VMEM((N_DEV - 1, m_per, n_per), jnp.int8),
            pltpu.SemaphoreType.DMA((N_DEV - 1,)),
            pltpu.SemaphoreType.DMA((N_DEV - 1,)),
        ],
        compiler_params=pltpu.CompilerParams(collective_id=0),
    )(x, w_mat)


# device time: 18859 ns/iter; 1.1772x vs baseline; 1.1772x over previous
import jax
import jax.numpy as jnp
from jax import lax
from jax.experimental import pallas as pl
from jax.experimental.pallas import tpu as pltpu

N_DEV = 4

_CLIP = 6.0
_QSCALE = 127.0 / _CLIP
_DEQ = _CLIP / 127.0

_ORDER = (2, 1, 3, 0)


def kernel(x, w_mat):
    m_per, k = x.shape
    n = w_mat.shape[1]
    n_per = n // N_DEV

    def body(x_hbm, w_hbm, out_hbm, x_vmem, w_buf, stage, send_buf, recv_buf,
             x_sem, w_sems, out_sems, send_sems, recv_sems):
        my = lax.axis_index("i")

        barrier_sem = pltpu.get_barrier_semaphore()
        for d in range(1, N_DEV):
            pl.semaphore_signal(
                barrier_sem, inc=1,
                device_id=((my + d) % N_DEV,),
                device_id_type=pl.DeviceIdType.MESH,
            )

        x_copy = pltpu.make_async_copy(x_hbm, x_vmem, x_sem)
        x_copy.start()

        def w_copy(s):
            j = (my + _ORDER[s]) % N_DEV
            return pltpu.make_async_copy(
                w_hbm.at[:, pl.ds(j * n_per, n_per)],
                w_buf.at[s],
                w_sems.at[s],
            )

        copies = [w_copy(s) for s in range(N_DEV)]
        copies[0].start()
        copies[1].start()

        pl.semaphore_wait(barrier_sem, N_DEV - 1)
        x_copy.wait()

        rdmas = []
        for s in range(N_DEV - 1):
            copies[s].wait()
            if s + 2 < N_DEV:
                copies[s + 2].start()
            chunk = jnp.dot(
                x_vmem[:, :], w_buf[s], preferred_element_type=jnp.float32
            )
            send_buf[s, :, :] = jnp.round(
                jnp.clip(chunk, -_CLIP, _CLIP) * _QSCALE
            ).astype(jnp.int8)
            rdma = pltpu.make_async_remote_copy(
                src_ref=send_buf.at[s],
                dst_ref=recv_buf.at[s],
                send_sem=send_sems.at[s],
                recv_sem=recv_sems.at[s],
                device_id=((my + _ORDER[s]) % N_DEV,),
                device_id_type=pl.DeviceIdType.MESH,
            )
            rdma.start()
            rdmas.append(rdma)

        def out_copy(s, origin):
            return pltpu.make_async_copy(
                stage.at[s],
                out_hbm.at[pl.ds(origin * m_per, m_per), :],
                out_sems.at[s],
            )

        copies[3].wait()
        stage[3, :, :] = jnp.dot(
            x_vmem[:, :], w_buf[3], preferred_element_type=jnp.float32
        )
        out_copies = [out_copy(3, my)]
        out_copies[0].start()

        for s in (1, 2, 0):
            rdmas[s].wait_recv()
            o = (my - _ORDER[s]) % N_DEV
            stage[s, :, :] = recv_buf[s, :, :].astype(jnp.float32) * _DEQ
            cp = out_copy(s, o)
            cp.start()
            out_copies.append(cp)

        for cp in out_copies:
            cp.wait()
        for s in range(N_DEV - 1):
            rdmas[s].wait_send()

    return pl.pallas_call(
        body,
        out_shape=jax.ShapeDtypeStruct((N_DEV * m_per, n_per), jnp.float32),
        in_specs=[
            pl.BlockSpec(memory_space=pl.ANY),
            pl.BlockSpec(memory_space=pl.ANY),
        ],
        out_specs=pl.BlockSpec(memory_space=pl.ANY),
        scratch_shapes=[
            pltpu.VMEM((m_per, k), jnp.float32),
            pltpu.VMEM((N_DEV, k, n_per), jnp.float32),
            pltpu.VMEM((N_DEV, m_per, n_per), jnp.float32),
            pltpu.VMEM((N_DEV - 1, m_per, n_per), jnp.int8),
            pltpu.VMEM((N_DEV - 1, m_per, n_per), jnp.int8),
            pltpu.SemaphoreType.DMA,
            pltpu.SemaphoreType.DMA((N_DEV,)),
            pltpu.SemaphoreType.DMA((N_DEV,)),
            pltpu.SemaphoreType.DMA((N_DEV - 1,)),
            pltpu.SemaphoreType.DMA((N_DEV - 1,)),
        ],
        compiler_params=pltpu.CompilerParams(collective_id=0),
    )(x, w_mat)


# device time: 18849 ns/iter; 1.1778x vs baseline; 1.0005x over previous
import jax
import jax.numpy as jnp
from jax import lax
from jax.experimental import pallas as pl
from jax.experimental.pallas import tpu as pltpu

N_DEV = 4

_CLIP = 6.0
_QSCALE = 127.0 / _CLIP
_DEQ = _CLIP / 127.0

_ORDER = (2, 1, 3, 0)


def kernel(x, w_mat):
    m_per, k = x.shape
    n = w_mat.shape[1]
    n_per = n // N_DEV

    def body(x_hbm, w_hbm, out_hbm, x_vmem, w_buf, stage, send_buf, recv_buf,
             x_sem, w_sems, out_sems, send_sems, recv_sems):
        my = lax.axis_index("i")

        barrier_sem = pltpu.get_barrier_semaphore()
        for d in range(1, N_DEV):
            pl.semaphore_signal(
                barrier_sem, inc=1,
                device_id=((my + d) % N_DEV,),
                device_id_type=pl.DeviceIdType.MESH,
            )

        x_copy = pltpu.make_async_copy(x_hbm, x_vmem, x_sem)
        x_copy.start()

        def w_copy(s):
            j = (my + _ORDER[s]) % N_DEV
            return pltpu.make_async_copy(
                w_hbm.at[:, pl.ds(j * n_per, n_per)],
                w_buf.at[s],
                w_sems.at[s],
            )

        copies = [w_copy(s) for s in range(N_DEV)]
        copies[0].start()
        copies[1].start()

        pl.semaphore_wait(barrier_sem, N_DEV - 1)
        x_copy.wait()

        rdmas = []
        for s in range(N_DEV - 1):
            copies[s].wait()
            if s + 2 < N_DEV:
                copies[s + 2].start()
            chunk = jnp.dot(
                x_vmem[:, :], w_buf[s], preferred_element_type=jnp.float32
            )
            send_buf[s, :, :] = jnp.round(
                jnp.clip(chunk, -_CLIP, _CLIP) * _QSCALE
            ).astype(jnp.int8)
            rdma = pltpu.make_async_remote_copy(
                src_ref=send_buf.at[s],
                dst_ref=recv_buf.at[s],
                send_sem=send_sems.at[s],
                recv_sem=recv_sems.at[s],
                device_id=((my + _ORDER[s]) % N_DEV,),
                device_id_type=pl.DeviceIdType.MESH,
            )
            rdma.start()
            rdmas.append(rdma)

        def out_copy(s, origin):
            return pltpu.make_async_copy(
                stage.at[s],
                out_hbm.at[pl.ds(origin * m_per, m_per), :],
                out_sems.at[s],
            )

        copies[3].wait()
        stage[3, :, :] = jnp.dot(
            x_vmem[:, :], w_buf[3], preferred_element_type=jnp.float32
        )
        out_copies = [out_copy(3, my)]
        out_copies[0].start()

        for s in (1, 2, 0):
            rdmas[s].wait_recv()
            o = (my - _ORDER[s]) % N_DEV
            stage[s, :, :] = recv_buf[s, :, :].astype(jnp.float32) * _DEQ
            cp = out_copy(s, o)
            cp.start()
            out_copies.append(cp)

        for cp in out_copies:
            cp.wait()
        for s in range(N_DEV - 1):
            rdmas[s].wait_send()

    return pl.pallas_call(
        body,
        out_shape=jax.ShapeDtypeStruct((N_DEV * m_per, n_per), jnp.float32),
        in_specs=[
            pl.BlockSpec(memory_space=pltpu.MemorySpace.HBM),
            pl.BlockSpec(memory_space=pltpu.MemorySpace.HBM),
        ],
        out_specs=pl.BlockSpec(memory_space=pltpu.MemorySpace.HBM),
        scratch_shapes=[
            pltpu.VMEM((m_per, k), jnp.float32),
            pltpu.VMEM((N_DEV, k, n_per), jnp.float32),
            pltpu.VMEM((N_DEV, m_per, n_per), jnp.float32),
            pltpu.VMEM((N_DEV - 1, m_per, n_per), jnp.int8),
            pltpu.VMEM((N_DEV - 1, m_per, n_per), jnp.int8),
            pltpu.SemaphoreType.DMA,
            pltpu.SemaphoreType.DMA((N_DEV,)),
            pltpu.SemaphoreType.DMA((N_DEV,)),
            pltpu.SemaphoreType.DMA((N_DEV - 1,)),
            pltpu.SemaphoreType.DMA((N_DEV - 1,)),
        ],
        compiler_params=pltpu.CompilerParams(collective_id=0),
    )(x, w_mat)
